# Initial kernel scaffold; baseline (speedup 1.0000x reference)
import jax
import jax.numpy as jnp
from jax.experimental import pallas as pl


def kernel(out, tgt):
    raise NotImplementedError("write your pallas kernel here")


# speedup vs baseline: 3.6945x; 3.6945x over previous
import functools

import jax
import jax.numpy as jnp
from jax.experimental import pallas as pl
from jax.experimental.pallas import tpu as pltpu

_EPS = 0.1
_IGNORE = 0


def _ls_ce_kernel(tgt_ref, x_ref, out_ref, acc_ref, *, num_classes, nblocks):
    i = pl.program_id(0)

    @pl.when(i == 0)
    def _init():
        acc_ref[0] = 0.0
        acc_ref[1] = 0.0

    x = x_ref[...]
    t = tgt_ref[...]
    r = x.shape[0]

    m = jnp.max(x, axis=1, keepdims=True)
    se = jnp.sum(jnp.exp(x - m), axis=1, keepdims=True)
    s = jnp.sum(x, axis=1, keepdims=True)
    cols = jax.lax.broadcasted_iota(jnp.int32, (r, num_classes), 1)
    g = jnp.sum(jnp.where(cols == t, x, 0.0), axis=1, keepdims=True)

    logz = m + jnp.log(se)
    fill = _EPS / (num_classes - 1)
    loss = fill * (num_classes * logz - s) + (1.0 - _EPS - fill) * (logz - g)
    valid = t != _IGNORE
    loss = jnp.where(valid, loss, 0.0)

    acc_ref[0] += jnp.sum(loss)
    acc_ref[1] += jnp.sum(valid.astype(jnp.float32))

    @pl.when(i == nblocks - 1)
    def _fin():
        out_ref[0, 0] = acc_ref[0] / acc_ref[1]


@jax.jit
def kernel(out, tgt):
    b, s, c = out.shape
    n = b * s
    rows_per_block = 256
    nblocks = n // rows_per_block

    x = out.reshape(n, c)
    t = tgt.reshape(n, 1)

    res = pl.pallas_call(
        functools.partial(_ls_ce_kernel, num_classes=c, nblocks=nblocks),
        grid=(nblocks,),
        in_specs=[
            pl.BlockSpec((rows_per_block, 1), lambda i: (i, 0)),
            pl.BlockSpec((rows_per_block, c), lambda i: (i, 0)),
        ],
        out_specs=pl.BlockSpec(
            (1, 1), lambda i: (0, 0), memory_space=pltpu.SMEM
        ),
        out_shape=jax.ShapeDtypeStruct((1, 1), jnp.float32),
        scratch_shapes=[pltpu.SMEM((2,), jnp.float32)],
        compiler_params=pltpu.CompilerParams(
            dimension_semantics=("arbitrary",),
        ),
    )(t, x)
    return res[0, 0]


# speedup vs baseline: 4.0528x; 1.0970x over previous
import functools

import jax
import jax.numpy as jnp
from jax.experimental import pallas as pl
from jax.experimental.pallas import tpu as pltpu

_EPS = 0.1
_IGNORE = 0


def _ls_ce_kernel(tgt_ref, x_ref, out_ref, acc_ref, *, num_classes, nblocks):
    i = pl.program_id(0)

    @pl.when(i == 0)
    def _init():
        acc_ref[0] = 0.0
        acc_ref[1] = 0.0

    x = x_ref[...]
    t = tgt_ref[...]
    r = x.shape[0]

    m = jnp.max(x, axis=1, keepdims=True)
    se = jnp.sum(jnp.exp(x - m), axis=1, keepdims=True)
    s = jnp.sum(x, axis=1, keepdims=True)
    cols = jax.lax.broadcasted_iota(jnp.int32, (r, num_classes), 1)
    g = jnp.sum(jnp.where(cols == t, x, 0.0), axis=1, keepdims=True)

    logz = m + jnp.log(se)
    fill = _EPS / (num_classes - 1)
    loss = fill * (num_classes * logz - s) + (1.0 - _EPS - fill) * (logz - g)
    valid = t != _IGNORE
    loss = jnp.where(valid, loss, 0.0)

    acc_ref[0] += jnp.sum(loss)
    acc_ref[1] += jnp.sum(valid.astype(jnp.float32))

    @pl.when(i == nblocks - 1)
    def _fin():
        out_ref[0, 0] = acc_ref[0] / acc_ref[1]


@jax.jit
def kernel(out, tgt):
    b, s, c = out.shape
    n = b * s
    rows_per_block = 512
    nblocks = n // rows_per_block

    x = out.reshape(n, c)
    t = tgt.reshape(n, 1)

    res = pl.pallas_call(
        functools.partial(_ls_ce_kernel, num_classes=c, nblocks=nblocks),
        grid=(nblocks,),
        in_specs=[
            pl.BlockSpec((rows_per_block, 1), lambda i: (i, 0)),
            pl.BlockSpec((rows_per_block, c), lambda i: (i, 0)),
        ],
        out_specs=pl.BlockSpec(
            (1, 1), lambda i: (0, 0), memory_space=pltpu.SMEM
        ),
        out_shape=jax.ShapeDtypeStruct((1, 1), jnp.float32),
        scratch_shapes=[pltpu.SMEM((2,), jnp.float32)],
        compiler_params=pltpu.CompilerParams(
            dimension_semantics=("arbitrary",),
        ),
    )(t, x)
    return res[0, 0]
